# Initial kernel scaffold; baseline (speedup 1.0000x reference)
import jax
import jax.numpy as jnp
from jax.experimental import pallas as pl


def kernel(feature, gt_boxes, im_info, conv_w, conv_b, cls_w, cls_b, reg_w, reg_b):
    raise NotImplementedError("write your pallas kernel here")


# speedup vs baseline: 8.9084x; 8.9084x over previous
import numpy as np
import jax
import jax.numpy as jnp
from jax import lax
from jax.experimental import pallas as pl

_ANCHOR_SCALES = (8.0, 16.0, 32.0)
_ANCHOR_RATIOS = (0.5, 1.0, 2.0)
_FEAT_STRIDE = 16
_PRE_NMS_TOP_N = 6000
_POST_NMS_TOP_N = 300
_NMS_THRESH = 0.7
_MIN_SIZE = 16.0

_H, _W, _A = 38, 50, 9
_N = _H * _W * _A
_ROWS = 136
_NP = _ROWS * 128


def _base_anchors(base_size=16):
    def whctrs(a):
        w = a[2] - a[0] + 1.0
        h = a[3] - a[1] + 1.0
        return w, h, a[0] + 0.5 * (w - 1.0), a[1] + 0.5 * (h - 1.0)

    def mk(ws, hs, x, y):
        return np.stack([x - 0.5 * (ws - 1.0), y - 0.5 * (hs - 1.0),
                         x + 0.5 * (ws - 1.0), y + 0.5 * (hs - 1.0)], axis=1)

    base = np.array([0.0, 0.0, base_size - 1.0, base_size - 1.0])
    w, h, x, y = whctrs(base)
    size = w * h
    ratios = np.array(_ANCHOR_RATIOS)
    ws = np.round(np.sqrt(size / ratios))
    hs = np.round(ws * ratios)
    ratio_anchors = mk(ws, hs, x, y)
    out = []
    for ra in ratio_anchors:
        w, h, x, y = whctrs(ra)
        scales = np.array(_ANCHOR_SCALES)
        out.append(mk(w * scales, h * scales, x, y))
    return np.concatenate(out, axis=0).astype(np.float32)


def _nms_body(s_ref, x1_ref, y1_ref, x2_ref, y2_ref, out_ref):
    score = s_ref[...]
    x1 = x1_ref[...]
    y1 = y1_ref[...]
    x2 = x2_ref[...]
    y2 = y2_ref[...]

    idx = (lax.broadcasted_iota(jnp.int32, (_ROWS, 128), 0) * 128
           + lax.broadcasted_iota(jnp.int32, (_ROWS, 128), 1))
    valid = idx < _N

    ib = lax.bitcast_convert_type(score, jnp.int32)
    ikey = ib ^ ((ib >> 31) & jnp.int32(0x7FFFFFFF))
    int_min = jnp.int32(-2147483648)
    key = jnp.where(valid, ikey, int_min)

    def kstep(i, tb):
        cand_b = tb | (jnp.int32(1) << (jnp.int32(31) - i))
        cand = cand_b ^ int_min
        cnt = jnp.sum((key >= cand).astype(jnp.int32))
        return jnp.where(cnt >= _PRE_NMS_TOP_N, cand_b, tb)

    kstar = lax.fori_loop(0, 32, kstep, jnp.int32(0)) ^ int_min
    cgt = jnp.sum((key > kstar).astype(jnp.int32))
    r = _PRE_NMS_TOP_N - cgt

    is_tie = key == kstar

    def istep(i, lohi):
        lo, hi = lohi
        mid = (lo + hi) // 2
        cnt = jnp.sum((is_tie & (idx <= mid)).astype(jnp.int32))
        take = cnt >= r
        return (jnp.where(take, lo, mid + 1), jnp.where(take, mid, hi))

    lo, hi = lax.fori_loop(0, 15, istep, (jnp.int32(0), jnp.int32(_N - 1)))
    istar = lo
    eligible = (key > kstar) | (is_tie & (idx <= istar))

    areas = (x2 - x1 + 1.0) * (y2 - y1 + 1.0)
    lane = lax.broadcasted_iota(jnp.int32, (1, 128), 1)

    def pick(p, alive):
        alive_b = alive > 0
        akey = jnp.where(alive_b, key, int_min)
        m = jnp.max(akey)
        found = m > int_min
        ism = alive_b & (akey == m)
        sel = jnp.min(jnp.where(ism, idx, jnp.int32(2**30)))
        onehot = idx == sel
        zf = jnp.float32(0.0)
        bx1 = jnp.sum(jnp.where(onehot, x1, zf))
        by1 = jnp.sum(jnp.where(onehot, y1, zf))
        bx2 = jnp.sum(jnp.where(onehot, x2, zf))
        by2 = jnp.sum(jnp.where(onehot, y2, zf))
        barea = (bx2 - bx1 + 1.0) * (by2 - by1 + 1.0)
        xx1 = jnp.maximum(bx1, x1)
        yy1 = jnp.maximum(by1, y1)
        xx2 = jnp.minimum(bx2, x2)
        yy2 = jnp.minimum(by2, y2)
        inter = (jnp.maximum(0.0, xx2 - xx1 + 1.0)
                 * jnp.maximum(0.0, yy2 - yy1 + 1.0))
        iou = inter / (barea + areas - inter)
        supp = iou > _NMS_THRESH
        alive = jnp.where(jnp.logical_and(supp, found), jnp.int32(0), alive)

        fmul = jnp.where(found, jnp.float32(1.0), jnp.float32(0.0))
        row = jnp.zeros((1, 128), jnp.float32)
        row = jnp.where(lane == 1, bx1 * fmul, row)
        row = jnp.where(lane == 2, by1 * fmul, row)
        row = jnp.where(lane == 3, bx2 * fmul, row)
        row = jnp.where(lane == 4, by2 * fmul, row)
        out_ref[pl.ds(p, 1), :] = row
        return alive

    lax.fori_loop(0, _POST_NMS_TOP_N, pick, eligible.astype(jnp.int32))


def _nms_pallas(scores, px1, py1, px2, py2, interpret=False):
    def pad2d(a):
        return jnp.pad(a, (0, _NP - _N)).reshape(_ROWS, 128)

    out = pl.pallas_call(
        _nms_body,
        out_shape=jax.ShapeDtypeStruct((_POST_NMS_TOP_N + 4, 128), jnp.float32),
        interpret=interpret,
    )(pad2d(scores), pad2d(px1), pad2d(py1), pad2d(px2), pad2d(py2))
    return out[:_POST_NMS_TOP_N, :5]


def _trunk_heads(feature, conv_w, conv_b, cls_w, cls_b, reg_w, reg_b):
    def conv(x, w, b, pad):
        y = lax.conv_general_dilated(x, w, (1, 1), pad,
                                     dimension_numbers=('NCHW', 'OIHW', 'NCHW'))
        return y + b[None, :, None, None]

    x = jax.nn.relu(conv(feature, conv_w, conv_b, 'SAME'))
    cls_score = conv(x, cls_w, cls_b, 'VALID')
    reg = conv(x, reg_w, reg_b, 'VALID')
    return cls_score, reg


def kernel(feature, gt_boxes, im_info, conv_w, conv_b, cls_w, cls_b, reg_w, reg_b):
    B = 1
    cls_score, reg = _trunk_heads(feature, conv_w, conv_b, cls_w, cls_b, reg_w, reg_b)

    cls_prob = jax.nn.softmax(
        cls_score.reshape(B, 2, _A, _H, _W), axis=1).reshape(B, 2 * _A, _H, _W)
    scores = jnp.transpose(cls_prob[0, _A:, :, :], (1, 2, 0)).reshape(-1)
    deltas = jnp.transpose(reg[0], (1, 2, 0)).reshape(-1, 4)

    anchors9 = jnp.asarray(_base_anchors())
    sx = jnp.arange(_W, dtype=jnp.float32) * _FEAT_STRIDE
    sy = jnp.arange(_H, dtype=jnp.float32) * _FEAT_STRIDE
    gx, gy = jnp.meshgrid(sx, sy)
    shifts = jnp.stack([gx.ravel(), gy.ravel(), gx.ravel(), gy.ravel()], axis=1)
    all_anchors = (anchors9[None, :, :] + shifts[:, None, :]).reshape(-1, 4)

    ws = all_anchors[:, 2] - all_anchors[:, 0] + 1.0
    hs = all_anchors[:, 3] - all_anchors[:, 1] + 1.0
    cx = all_anchors[:, 0] + 0.5 * ws
    cy = all_anchors[:, 1] + 0.5 * hs
    px = deltas[:, 0] * ws + cx
    py = deltas[:, 1] * hs + cy
    pw = jnp.exp(deltas[:, 2]) * ws
    ph = jnp.exp(deltas[:, 3]) * hs
    x1 = jnp.clip(px - 0.5 * pw, 0.0, im_info[0, 1] - 1.0)
    y1 = jnp.clip(py - 0.5 * ph, 0.0, im_info[0, 0] - 1.0)
    x2 = jnp.clip(px + 0.5 * pw, 0.0, im_info[0, 1] - 1.0)
    y2 = jnp.clip(py + 0.5 * ph, 0.0, im_info[0, 0] - 1.0)

    min_size = _MIN_SIZE * im_info[0, 2]
    valid = jnp.logical_and(x2 - x1 + 1.0 >= min_size, y2 - y1 + 1.0 >= min_size)
    scores = jnp.where(valid, scores, -jnp.inf)

    return _nms_pallas(scores, x1, y1, x2, y2)


# speedup vs baseline: 8.9633x; 1.0062x over previous
import numpy as np
import jax
import jax.numpy as jnp
from jax import lax
from jax.experimental import pallas as pl
from jax.experimental.pallas import tpu as pltpu

_ANCHOR_SCALES = (8.0, 16.0, 32.0)
_ANCHOR_RATIOS = (0.5, 1.0, 2.0)
_FEAT_STRIDE = 16
_PRE_NMS_TOP_N = 6000
_POST_NMS_TOP_N = 300
_NMS_THRESH = 0.7
_MIN_SIZE = 16.0

_H, _W, _A = 38, 50, 9
_N = _H * _W * _A
_ROWS = 134
_NP = _ROWS * 128


def _base_anchors(base_size=16):
    def whctrs(a):
        w = a[2] - a[0] + 1.0
        h = a[3] - a[1] + 1.0
        return w, h, a[0] + 0.5 * (w - 1.0), a[1] + 0.5 * (h - 1.0)

    def mk(ws, hs, x, y):
        return np.stack([x - 0.5 * (ws - 1.0), y - 0.5 * (hs - 1.0),
                         x + 0.5 * (ws - 1.0), y + 0.5 * (hs - 1.0)], axis=1)

    base = np.array([0.0, 0.0, base_size - 1.0, base_size - 1.0])
    w, h, x, y = whctrs(base)
    size = w * h
    ratios = np.array(_ANCHOR_RATIOS)
    ws = np.round(np.sqrt(size / ratios))
    hs = np.round(ws * ratios)
    ratio_anchors = mk(ws, hs, x, y)
    out = []
    for ra in ratio_anchors:
        w, h, x, y = whctrs(ra)
        scales = np.array(_ANCHOR_SCALES)
        out.append(mk(w * scales, h * scales, x, y))
    return np.concatenate(out, axis=0).astype(np.float32)


def _nms_body(s_ref, x1_ref, y1_ref, x2_ref, y2_ref, out_ref, key_ref, area_ref):
    score = s_ref[...]

    idx = (lax.broadcasted_iota(jnp.int32, (_ROWS, 128), 0) * 128
           + lax.broadcasted_iota(jnp.int32, (_ROWS, 128), 1))
    valid = idx < _N

    ib = lax.bitcast_convert_type(score, jnp.int32)
    ikey = ib ^ ((ib >> 31) & jnp.int32(0x7FFFFFFF))
    int_min = jnp.int32(-2147483648)
    key = jnp.where(valid, ikey, int_min)

    def kstep(i, tb):
        cand_b = tb | (jnp.int32(1) << (jnp.int32(31) - i))
        cand = cand_b ^ int_min
        cnt = jnp.sum((key >= cand).astype(jnp.int32))
        return jnp.where(cnt >= _PRE_NMS_TOP_N, cand_b, tb)

    kstar = lax.fori_loop(0, 32, kstep, jnp.int32(0)) ^ int_min
    cgt = jnp.sum((key > kstar).astype(jnp.int32))
    r = _PRE_NMS_TOP_N - cgt

    is_tie = key == kstar

    def istep(i, lohi):
        lo, hi = lohi
        mid = (lo + hi) // 2
        cnt = jnp.sum((is_tie & (idx <= mid)).astype(jnp.int32))
        take = cnt >= r
        return (jnp.where(take, lo, mid + 1), jnp.where(take, mid, hi))

    lo, hi = lax.fori_loop(0, 15, istep, (jnp.int32(0), jnp.int32(_N - 1)))
    istar = lo
    eligible = (key > kstar) | (is_tie & (idx <= istar))

    key_ref[...] = jnp.where(eligible, key, int_min)
    area_ref[...] = ((x2_ref[...] - x1_ref[...] + 1.0)
                     * (y2_ref[...] - y1_ref[...] + 1.0))
    lane = lax.broadcasted_iota(jnp.int32, (1, 128), 1)

    def extract(ref, r, c):
        row = ref[pl.ds(r, 1), :]
        return jnp.sum(jnp.where(lane == c, row, jnp.float32(0.0)))

    def pick(p, carry):
        akey = key_ref[...]
        m = jnp.max(akey)
        found = m > int_min
        sel = jnp.min(jnp.where(akey == m, idx, jnp.int32(2**30)))
        r = sel // 128
        c = sel - r * 128

        @pl.when(found)
        def _():
            bx1 = extract(x1_ref, r, c)
            by1 = extract(y1_ref, r, c)
            bx2 = extract(x2_ref, r, c)
            by2 = extract(y2_ref, r, c)
            barea = (bx2 - bx1 + 1.0) * (by2 - by1 + 1.0)
            xx1 = jnp.maximum(bx1, x1_ref[...])
            yy1 = jnp.maximum(by1, y1_ref[...])
            xx2 = jnp.minimum(bx2, x2_ref[...])
            yy2 = jnp.minimum(by2, y2_ref[...])
            inter = (jnp.maximum(0.0, xx2 - xx1 + 1.0)
                     * jnp.maximum(0.0, yy2 - yy1 + 1.0))
            iou = inter / (barea + area_ref[...] - inter)
            key_ref[...] = jnp.where(iou > _NMS_THRESH, int_min, akey)

            row = jnp.zeros((1, 128), jnp.float32)
            row = jnp.where(lane == 1, bx1, row)
            row = jnp.where(lane == 2, by1, row)
            row = jnp.where(lane == 3, bx2, row)
            row = jnp.where(lane == 4, by2, row)
            out_ref[pl.ds(p, 1), :] = row

        @pl.when(jnp.logical_not(found))
        def _():
            out_ref[pl.ds(p, 1), :] = jnp.zeros((1, 128), jnp.float32)

        return carry

    lax.fori_loop(0, _POST_NMS_TOP_N, pick, jnp.int32(0))


def _nms_pallas(scores, px1, py1, px2, py2, interpret=False):
    def pad2d(a):
        return jnp.pad(a, (0, _NP - _N)).reshape(_ROWS, 128)

    out = pl.pallas_call(
        _nms_body,
        out_shape=jax.ShapeDtypeStruct((_POST_NMS_TOP_N + 4, 128), jnp.float32),
        scratch_shapes=[pltpu.VMEM((_ROWS, 128), jnp.int32),
                        pltpu.VMEM((_ROWS, 128), jnp.float32)],
        interpret=interpret,
    )(pad2d(scores), pad2d(px1), pad2d(py1), pad2d(px2), pad2d(py2))
    return out[:_POST_NMS_TOP_N, :5]


def _trunk_heads(feature, conv_w, conv_b, cls_w, cls_b, reg_w, reg_b):
    def conv(x, w, b, pad):
        y = lax.conv_general_dilated(x, w, (1, 1), pad,
                                     dimension_numbers=('NCHW', 'OIHW', 'NCHW'))
        return y + b[None, :, None, None]

    x = jax.nn.relu(conv(feature, conv_w, conv_b, 'SAME'))
    cls_score = conv(x, cls_w, cls_b, 'VALID')
    reg = conv(x, reg_w, reg_b, 'VALID')
    return cls_score, reg


def kernel(feature, gt_boxes, im_info, conv_w, conv_b, cls_w, cls_b, reg_w, reg_b):
    B = 1
    cls_score, reg = _trunk_heads(feature, conv_w, conv_b, cls_w, cls_b, reg_w, reg_b)

    cls_prob = jax.nn.softmax(
        cls_score.reshape(B, 2, _A, _H, _W), axis=1).reshape(B, 2 * _A, _H, _W)
    scores = jnp.transpose(cls_prob[0, _A:, :, :], (1, 2, 0)).reshape(-1)
    deltas = jnp.transpose(reg[0], (1, 2, 0)).reshape(-1, 4)

    anchors9 = jnp.asarray(_base_anchors())
    sx = jnp.arange(_W, dtype=jnp.float32) * _FEAT_STRIDE
    sy = jnp.arange(_H, dtype=jnp.float32) * _FEAT_STRIDE
    gx, gy = jnp.meshgrid(sx, sy)
    shifts = jnp.stack([gx.ravel(), gy.ravel(), gx.ravel(), gy.ravel()], axis=1)
    all_anchors = (anchors9[None, :, :] + shifts[:, None, :]).reshape(-1, 4)

    ws = all_anchors[:, 2] - all_anchors[:, 0] + 1.0
    hs = all_anchors[:, 3] - all_anchors[:, 1] + 1.0
    cx = all_anchors[:, 0] + 0.5 * ws
    cy = all_anchors[:, 1] + 0.5 * hs
    px = deltas[:, 0] * ws + cx
    py = deltas[:, 1] * hs + cy
    pw = jnp.exp(deltas[:, 2]) * ws
    ph = jnp.exp(deltas[:, 3]) * hs
    x1 = jnp.clip(px - 0.5 * pw, 0.0, im_info[0, 1] - 1.0)
    y1 = jnp.clip(py - 0.5 * ph, 0.0, im_info[0, 0] - 1.0)
    x2 = jnp.clip(px + 0.5 * pw, 0.0, im_info[0, 1] - 1.0)
    y2 = jnp.clip(py + 0.5 * ph, 0.0, im_info[0, 0] - 1.0)

    min_size = _MIN_SIZE * im_info[0, 2]
    valid = jnp.logical_and(x2 - x1 + 1.0 >= min_size, y2 - y1 + 1.0 >= min_size)
    scores = jnp.where(valid, scores, -jnp.inf)

    return _nms_pallas(scores, x1, y1, x2, y2)
